# baseline (speedup 1.0000x reference)
import functools

import jax
import jax.numpy as jnp
from jax import lax
from jax.experimental import pallas as pl
from jax.experimental.pallas import tpu as pltpu
from jax.experimental.pallas import tpu_sc as plsc

K = 20
R = 512
PADD = 16
INF = float("inf")


def _topk_body(n_keys, xyz_ref, xyzT_ref, src_ref, srcoff_ref):
    b = pl.program_id(0)
    q = xyz_ref[0]
    keys = xyzT_ref[0]
    xx = jnp.sum(q * q, axis=1, keepdims=True)
    yy = jnp.sum(keys * keys, axis=0, keepdims=True)
    inner = jnp.dot(q, keys, preferred_element_type=jnp.float32)
    dist = jnp.maximum(xx + yy - 2.0 * inner, 0.0)
    iota = lax.broadcasted_iota(jnp.int32, (R, n_keys), 1)
    big = jnp.int32(n_keys)
    idxs = []
    for j in range(K + 1):
        v = jnp.min(dist, axis=1, keepdims=True)
        eq = dist == v
        idx = jnp.min(jnp.where(eq, iota, big), axis=1, keepdims=True)
        if j > 0:
            idxs.append(idx)
        if j < K:
            dist = jnp.where(iota == idx, INF, dist)
    ind = jnp.concatenate(idxs, axis=1)
    src_ref[0] = ind
    srcoff_ref[0] = ind + b * n_keys


def _topk(xyz):
    B, N, _ = xyz.shape
    xyzT = jnp.transpose(xyz, (0, 2, 1))
    return pl.pallas_call(
        functools.partial(_topk_body, N),
        grid=(B, N // R),
        in_specs=[
            pl.BlockSpec((1, R, 3), lambda b, i: (b, i, 0)),
            pl.BlockSpec((1, 3, N), lambda b, i: (b, 0, 0)),
        ],
        out_specs=[
            pl.BlockSpec((1, R, K), lambda b, i: (b, i, 0)),
            pl.BlockSpec((1, R, K), lambda b, i: (b, i, 0)),
        ],
        out_shape=[
            jax.ShapeDtypeStruct((B, N, K), jnp.int32),
            jax.ShapeDtypeStruct((B, N, K), jnp.int32),
        ],
    )(xyz, xyzT)


def _edge_gather(pos_pad, src_off, dst_off):
    E = src_off.shape[0]
    info = plsc.get_sparse_core_info()
    NC, NS = info.num_cores, info.num_subcores
    NW = NC * NS
    e_per_w = E // NW
    CH = 512
    n_ch = e_per_w // CH
    mesh = plsc.VectorSubcoreMesh(core_axis_name="c", subcore_axis_name="s")

    @functools.partial(
        pl.kernel, mesh=mesh,
        compiler_params=pltpu.CompilerParams(use_tc_tiling_on_sc=False),
        out_type=jax.ShapeDtypeStruct((E, PADD), jnp.float32),
        scratch_types=[
            pltpu.VMEM((CH,), jnp.int32),
            pltpu.VMEM((CH,), jnp.int32),
            pltpu.VMEM((CH, PADD), jnp.float32),
            pltpu.VMEM((CH, PADD), jnp.float32),
            pltpu.VMEM((CH, PADD), jnp.float32),
            pltpu.SemaphoreType.DMA,
        ],
    )
    def k(pos_hbm, sidx_hbm, didx_hbm, out_hbm,
          sidx_v, didx_v, srow_v, drow_v, out_v, sem):
        wid = lax.axis_index("s") * NC + lax.axis_index("c")

        def chunk_body(c, carry):
            base = wid * e_per_w + c * CH
            pltpu.sync_copy(sidx_hbm.at[pl.ds(base, CH)], sidx_v)
            pltpu.sync_copy(didx_hbm.at[pl.ds(base, CH)], didx_v)
            pltpu.async_copy(pos_hbm.at[sidx_v], srow_v, sem).wait()
            pltpu.async_copy(pos_hbm.at[didx_v], drow_v, sem).wait()

            def e_body(e, c2):
                out_v[e] = drow_v[e] - srow_v[e]
                return c2

            lax.fori_loop(0, CH, e_body, 0)
            pltpu.sync_copy(out_v, out_hbm.at[pl.ds(base, CH)])
            return carry

        lax.fori_loop(0, n_ch, chunk_body, 0)

    return k(pos_pad, src_off, dst_off)


def kernel(xyz):
    B, N, _ = xyz.shape
    pos = xyz
    ind, ind_off = _topk(xyz)
    src = ind.reshape(B, N * K)

    dst = jnp.repeat(jnp.arange(N, dtype=jnp.int32), K)
    dst_b = jnp.tile(dst[None, :], (B, 1))

    pos_pad = jnp.pad(pos.reshape(B * N, 3), ((0, 0), (0, PADD - 3)))
    src_off = ind_off.reshape(B * N * K)
    dst_off = (dst_b + jnp.arange(B, dtype=jnp.int32)[:, None] * N).reshape(-1)
    d_pad = _edge_gather(pos_pad, src_off, dst_off)
    d = d_pad[:, :3].reshape(B, N * K, 3)

    f = jnp.ones((B, N, 1, 1), dtype=jnp.float32)
    return (pos, f, src, dst_b, d)


# speedup vs baseline: 1.0449x; 1.0449x over previous
import functools

import jax
import jax.numpy as jnp
from jax import lax
from jax.experimental import pallas as pl
from jax.experimental.pallas import tpu as pltpu
from jax.experimental.pallas import tpu_sc as plsc

K = 20
R = 128
PADD = 16
INF = float("inf")


def _topk_body(n_keys, xyz_ref, xyzT_ref, src_ref, srcoff_ref):
    b = pl.program_id(0)
    q = xyz_ref[0]
    keys = xyzT_ref[0]
    xx = jnp.sum(q * q, axis=1, keepdims=True)
    yy = jnp.sum(keys * keys, axis=0, keepdims=True)
    inner = jnp.dot(q, keys, preferred_element_type=jnp.float32)
    dist = jnp.maximum(xx + yy - 2.0 * inner, 0.0)
    iota = lax.broadcasted_iota(jnp.int32, (R, n_keys), 1)
    big = jnp.int32(n_keys)
    idxs = []
    for j in range(K + 1):
        v = jnp.min(dist, axis=1, keepdims=True)
        eq = dist == v
        idx = jnp.min(jnp.where(eq, iota, big), axis=1, keepdims=True)
        if j > 0:
            idxs.append(idx)
        if j < K:
            dist = jnp.where(iota == idx, INF, dist)
    ind = jnp.concatenate(idxs, axis=1)
    src_ref[0] = ind
    srcoff_ref[0] = ind + b * n_keys


def _topk(xyz):
    B, N, _ = xyz.shape
    xyzT = jnp.transpose(xyz, (0, 2, 1))
    return pl.pallas_call(
        functools.partial(_topk_body, N),
        grid=(B, N // R),
        in_specs=[
            pl.BlockSpec((1, R, 3), lambda b, i: (b, i, 0)),
            pl.BlockSpec((1, 3, N), lambda b, i: (b, 0, 0)),
        ],
        out_specs=[
            pl.BlockSpec((1, R, K), lambda b, i: (b, i, 0)),
            pl.BlockSpec((1, R, K), lambda b, i: (b, i, 0)),
        ],
        out_shape=[
            jax.ShapeDtypeStruct((B, N, K), jnp.int32),
            jax.ShapeDtypeStruct((B, N, K), jnp.int32),
        ],
    )(xyz, xyzT)


def _edge_gather(pos_pad, src_off, dst_off):
    E = src_off.shape[0]
    info = plsc.get_sparse_core_info()
    NC, NS = info.num_cores, info.num_subcores
    NW = NC * NS
    e_per_w = E // NW
    CH = 512
    n_ch = e_per_w // CH
    mesh = plsc.VectorSubcoreMesh(core_axis_name="c", subcore_axis_name="s")

    @functools.partial(
        pl.kernel, mesh=mesh,
        compiler_params=pltpu.CompilerParams(use_tc_tiling_on_sc=False),
        out_type=jax.ShapeDtypeStruct((E, PADD), jnp.float32),
        scratch_types=[
            pltpu.VMEM((CH,), jnp.int32),
            pltpu.VMEM((CH,), jnp.int32),
            pltpu.VMEM((CH, PADD), jnp.float32),
            pltpu.VMEM((CH, PADD), jnp.float32),
            pltpu.VMEM((CH, PADD), jnp.float32),
            pltpu.SemaphoreType.DMA,
        ],
    )
    def k(pos_hbm, sidx_hbm, didx_hbm, out_hbm,
          sidx_v, didx_v, srow_v, drow_v, out_v, sem):
        wid = lax.axis_index("s") * NC + lax.axis_index("c")

        def chunk_body(c, carry):
            base = wid * e_per_w + c * CH
            pltpu.sync_copy(sidx_hbm.at[pl.ds(base, CH)], sidx_v)
            pltpu.sync_copy(didx_hbm.at[pl.ds(base, CH)], didx_v)
            pltpu.async_copy(pos_hbm.at[sidx_v], srow_v, sem).wait()
            pltpu.async_copy(pos_hbm.at[didx_v], drow_v, sem).wait()

            def e_body(e, c2):
                out_v[e] = drow_v[e] - srow_v[e]
                return c2

            lax.fori_loop(0, CH, e_body, 0)
            pltpu.sync_copy(out_v, out_hbm.at[pl.ds(base, CH)])
            return carry

        lax.fori_loop(0, n_ch, chunk_body, 0)

    return k(pos_pad, src_off, dst_off)


def kernel(xyz):
    B, N, _ = xyz.shape
    pos = xyz
    ind, ind_off = _topk(xyz)
    src = ind.reshape(B, N * K)

    dst = jnp.repeat(jnp.arange(N, dtype=jnp.int32), K)
    dst_b = jnp.tile(dst[None, :], (B, 1))

    pos_pad = jnp.pad(pos.reshape(B * N, 3), ((0, 0), (0, PADD - 3)))
    src_off = ind_off.reshape(B * N * K)
    dst_off = (dst_b + jnp.arange(B, dtype=jnp.int32)[:, None] * N).reshape(-1)
    d_pad = _edge_gather(pos_pad, src_off, dst_off)
    d = d_pad[:, :3].reshape(B, N * K, 3)

    f = jnp.ones((B, N, 1, 1), dtype=jnp.float32)
    return (pos, f, src, dst_b, d)


# speedup vs baseline: 1.1485x; 1.0992x over previous
import functools

import jax
import jax.numpy as jnp
from jax import lax
from jax.experimental import pallas as pl
from jax.experimental.pallas import tpu as pltpu
from jax.experimental.pallas import tpu_sc as plsc

K = 20
R = 256
PADD = 16
INF = float("inf")


def _topk_body(n_keys, xyz_ref, xyzT_ref, src_ref, srcoff_ref):
    b = pl.program_id(0)
    q = xyz_ref[0]
    keys = xyzT_ref[0]
    xx = jnp.sum(q * q, axis=1, keepdims=True)
    yy = jnp.sum(keys * keys, axis=0, keepdims=True)
    inner = jnp.dot(q, keys, preferred_element_type=jnp.float32)
    dist = jnp.maximum(xx + yy - 2.0 * inner, 0.0)
    iota = lax.broadcasted_iota(jnp.int32, (R, n_keys), 1)
    big = jnp.int32(n_keys)
    idxs = []
    for j in range(K + 1):
        v = jnp.min(dist, axis=1, keepdims=True)
        eq = dist == v
        idx = jnp.min(jnp.where(eq, iota, big), axis=1, keepdims=True)
        if j > 0:
            idxs.append(idx)
        if j < K:
            dist = jnp.where(iota == idx, INF, dist)
    ind = jnp.concatenate(idxs, axis=1)
    src_ref[0] = ind
    srcoff_ref[0] = ind + b * n_keys


def _topk(xyz):
    B, N, _ = xyz.shape
    xyzT = jnp.transpose(xyz, (0, 2, 1))
    return pl.pallas_call(
        functools.partial(_topk_body, N),
        grid=(B, N // R),
        in_specs=[
            pl.BlockSpec((1, R, 3), lambda b, i: (b, i, 0)),
            pl.BlockSpec((1, 3, N), lambda b, i: (b, 0, 0)),
        ],
        out_specs=[
            pl.BlockSpec((1, R, K), lambda b, i: (b, i, 0)),
            pl.BlockSpec((1, R, K), lambda b, i: (b, i, 0)),
        ],
        out_shape=[
            jax.ShapeDtypeStruct((B, N, K), jnp.int32),
            jax.ShapeDtypeStruct((B, N, K), jnp.int32),
        ],
    )(xyz, xyzT)


def _edge_gather(pos_pad, src_off, dst_off):
    E = src_off.shape[0]
    info = plsc.get_sparse_core_info()
    NC, NS = info.num_cores, info.num_subcores
    NW = NC * NS
    e_per_w = E // NW
    CH = 512
    n_ch = e_per_w // CH
    mesh = plsc.VectorSubcoreMesh(core_axis_name="c", subcore_axis_name="s")

    @functools.partial(
        pl.kernel, mesh=mesh,
        compiler_params=pltpu.CompilerParams(use_tc_tiling_on_sc=False),
        out_type=jax.ShapeDtypeStruct((E, PADD), jnp.float32),
        scratch_types=[
            pltpu.VMEM((CH,), jnp.int32),
            pltpu.VMEM((CH,), jnp.int32),
            pltpu.VMEM((CH, PADD), jnp.float32),
            pltpu.VMEM((CH, PADD), jnp.float32),
            pltpu.VMEM((CH, PADD), jnp.float32),
            pltpu.SemaphoreType.DMA,
        ],
    )
    def k(pos_hbm, sidx_hbm, didx_hbm, out_hbm,
          sidx_v, didx_v, srow_v, drow_v, out_v, sem):
        wid = lax.axis_index("s") * NC + lax.axis_index("c")

        def chunk_body(c, carry):
            base = wid * e_per_w + c * CH
            pltpu.sync_copy(sidx_hbm.at[pl.ds(base, CH)], sidx_v)
            pltpu.sync_copy(didx_hbm.at[pl.ds(base, CH)], didx_v)
            pltpu.async_copy(pos_hbm.at[sidx_v], srow_v, sem).wait()
            pltpu.async_copy(pos_hbm.at[didx_v], drow_v, sem).wait()

            def e_body(e, c2):
                out_v[e] = drow_v[e] - srow_v[e]
                return c2

            lax.fori_loop(0, CH, e_body, 0)
            pltpu.sync_copy(out_v, out_hbm.at[pl.ds(base, CH)])
            return carry

        lax.fori_loop(0, n_ch, chunk_body, 0)

    return k(pos_pad, src_off, dst_off)


def kernel(xyz):
    B, N, _ = xyz.shape
    pos = xyz
    ind, ind_off = _topk(xyz)
    src = ind.reshape(B, N * K)

    dst = jnp.repeat(jnp.arange(N, dtype=jnp.int32), K)
    dst_b = jnp.tile(dst[None, :], (B, 1))

    pos_pad = jnp.pad(pos.reshape(B * N, 3), ((0, 0), (0, PADD - 3)))
    src_off = ind_off.reshape(B * N * K)
    dst_off = (dst_b + jnp.arange(B, dtype=jnp.int32)[:, None] * N).reshape(-1)
    d_pad = _edge_gather(pos_pad, src_off, dst_off)
    d = d_pad[:, :3].reshape(B, N * K, 3)

    f = jnp.ones((B, N, 1, 1), dtype=jnp.float32)
    return (pos, f, src, dst_b, d)
